# baseline (speedup 1.0000x reference)
import functools

import jax
import jax.numpy as jnp
from jax import lax
from jax.experimental import pallas as pl
from jax.experimental.pallas import tpu as pltpu
from jax.experimental.pallas import tpu_sc as plsc

NU = 25000
NI = 25000
NN = NU + NI
D = 64
NC = 2
NS = 16
H = 25000
HP = 25088
ROWS_T = HP // NS
CH = 128
INNER = 14
SUP = CH * INNER
NSUP_A = 28
NSUP_B = 7
EA = NS * NSUP_A * SUP
EB = NC * NS * NSUP_B * SUP
ZR = 28
EBUF = SUP + 272


def _sc_body_a(pre, nrow, ncol, nval,
               e_out,
               rowb, colb, valb, idx_g0, idx_s0, vsc0, idx_g1, idx_s1, vsc1,
               gb0, gb1, zbuf, acc,
               sem0, sem1, sem_s0, sem_s1):
    cid = lax.axis_index("c")
    sid = lax.axis_index("s")
    ii16 = lax.broadcasted_iota(jnp.int32, (16,), 0)
    lanes = [ii16 * 0 + e for e in range(16)]

    def zrow(r, carry):
        for j in range(D // 16):
            zbuf[r, pl.ds(j * 16, 16)] = jnp.zeros((16,), jnp.float32)
        return carry

    lax.fori_loop(0, ZR, zrow, 0)

    def zero_acc():
        zbase = sid * ROWS_T
        for z in range(ROWS_T // ZR):
            pltpu.sync_copy(zbuf, acc.at[pl.ds(zbase + z * ZR, ZR)])
        plsc.subcore_barrier()

    def writeback(out_h):
        plsc.subcore_barrier()
        pltpu.sync_copy(acc.at[pl.ds(sid * ROWS_T, ROWS_T)],
                        out_h.at[cid, pl.ds(sid * ROWS_T, ROWS_T)])
        plsc.subcore_barrier()

    def scale_chunk(gb, vs, voff):
        def scale_body(g, carry3):
            v16 = vs[pl.ds(voff + g * 16, 16)]
            base = g * 16
            for e in range(16):
                bc = v16.at[lanes[e]].get(mode="promise_in_bounds")
                for j in range(D // 16):
                    sl = pl.ds(j * 16, 16)
                    gb[base + e, sl] = gb[base + e, sl] * bc
            return carry3

        lax.fori_loop(0, CH // 16, scale_body, 0)

    def wait_scatter(gb, isc, sem_s):
        pltpu.make_async_copy(gb, acc.at[isc], sem_s).wait()

    def run_phase_a(row_h, col_h, val_h, n_super, out_h):
        zero_acc()
        edge_base = sid * (n_super * SUP)
        lo = cid * H
        shidx = [jnp.maximum(ii16 - k, 0) for k in (1, 2, 4, 8)]
        shmask = [ii16 >= k for k in (1, 2, 4, 8)]

        def prep_idx(isc, ig, off):
            for j in range(CH // 16):
                isc[pl.ds(j * 16, 16)] = rowb[pl.ds(off + j * 16, 16)]
                ig[pl.ds(j * 16, 16)] = colb[pl.ds(off + j * 16, 16)]

        def super_body(g, carry):
            sb = edge_base + g * SUP
            pltpu.sync_copy(row_h.at[pl.ds(sb, EBUF)], rowb)
            pltpu.sync_copy(col_h.at[pl.ds(sb, EBUF)], colb)
            pltpu.sync_copy(val_h.at[pl.ds(sb, EBUF)], valb)

            def fbody(j, p):
                slb = pl.ds(j * 16, 16)
                r = rowb[slb]
                cc = colb[slb]
                vv = valb[slb]
                m = r - jnp.where(r >= H, H, 0)
                ok = (r >= lo) & (r < lo + H)
                s = jnp.where(ok, 1, 0)
                for t in range(4):
                    sh = s.at[shidx[t]].get(mode="promise_in_bounds")
                    s = s + jnp.where(shmask[t], sh, 0)
                sel = ii16 * 0
                for step in (8, 4, 2, 1):
                    sv = s.at[sel + (step - 1)].get(mode="promise_in_bounds")
                    sel = jnp.where(sv < ii16 + 1, sel + step, sel)
                slw = pl.ds(p, 16)
                rowb[slw] = m.at[sel].get(mode="promise_in_bounds")
                colb[slw] = cc.at[sel].get(mode="promise_in_bounds")
                valb[slw] = vv.at[sel].get(mode="promise_in_bounds")
                return p + s[15]

            p = lax.fori_loop(0, SUP // 16, fbody, 0)
            for j in range(16):
                slp = pl.ds(p + j * 16, 16)
                rowb[slp] = ii16 + (j * 16)
                colb[slp] = ii16 + (j * 16)
                valb[slp] = jnp.zeros((16,), jnp.float32)
            npair = (p + 255) // 256

            @pl.when(npair > 0)
            def _():
                prep_idx(idx_s0, idx_g0, 0)
                pltpu.async_copy(pre.at[idx_g0], gb0, sem0)

            def pair_body(i, carry2):
                c0 = i * 256
                c1 = c0 + CH

                @pl.when(i > 0)
                def _():
                    wait_scatter(gb1, idx_s1, sem_s1)

                prep_idx(idx_s1, idx_g1, c1)
                pltpu.async_copy(pre.at[idx_g1], gb1, sem1)
                pltpu.make_async_copy(pre.at[idx_g0], gb0, sem0).wait()
                scale_chunk(gb0, valb, c0)
                pltpu.async_copy(gb0, acc.at[idx_s0], sem_s0, add=True)

                @pl.when(i < npair - 1)
                def _():
                    wait_scatter(gb0, idx_s0, sem_s0)
                    prep_idx(idx_s0, idx_g0, c0 + 256)
                    pltpu.async_copy(pre.at[idx_g0], gb0, sem0)

                pltpu.make_async_copy(pre.at[idx_g1], gb1, sem1).wait()
                scale_chunk(gb1, valb, c1)
                pltpu.async_copy(gb1, acc.at[idx_s1], sem_s1, add=True)
                return carry2

            lax.fori_loop(0, npair, pair_body, 0)

            @pl.when(npair > 0)
            def _():
                wait_scatter(gb0, idx_s0, sem_s0)
                wait_scatter(gb1, idx_s1, sem_s1)

            return carry

        lax.fori_loop(0, n_super, super_body, 0)
        writeback(out_h)

    def run_phase(row_h, col_h, val_h, n_super, col_off, out_h):
        zero_acc()
        edge_base = (cid * NS + sid) * (n_super * SUP)

        def compute_idx(ci, ig, isc, vs):
            cb = ci * CH
            for j in range(CH // 16):
                sl16 = pl.ds(j * 16, 16)
                slb = pl.ds(cb + j * 16, 16)
                isc[sl16] = rowb[slb]
                ig[sl16] = colb[slb] + col_off
                vs[sl16] = valb[slb]

        def scale_scatter(gb, vs, isc, sem_s):
            scale_chunk(gb, vs, 0)
            pltpu.async_copy(gb, acc.at[isc], sem_s, add=True)

        def super_body(g, carry):
            sb = edge_base + g * SUP
            pltpu.sync_copy(row_h.at[pl.ds(sb, EBUF)], rowb)
            pltpu.sync_copy(col_h.at[pl.ds(sb, EBUF)], colb)
            pltpu.sync_copy(val_h.at[pl.ds(sb, EBUF)], valb)

            compute_idx(0, idx_g0, idx_s0, vsc0)
            pltpu.async_copy(pre.at[idx_g0], gb0, sem0)

            def pair_body(h, carry2):
                @pl.when(h > 0)
                def _():
                    wait_scatter(gb1, idx_s1, sem_s1)

                compute_idx(2 * h + 1, idx_g1, idx_s1, vsc1)
                pltpu.async_copy(pre.at[idx_g1], gb1, sem1)
                pltpu.make_async_copy(pre.at[idx_g0], gb0, sem0).wait()
                scale_scatter(gb0, vsc0, idx_s0, sem_s0)

                @pl.when(h < INNER // 2 - 1)
                def _():
                    wait_scatter(gb0, idx_s0, sem_s0)
                    compute_idx(2 * h + 2, idx_g0, idx_s0, vsc0)
                    pltpu.async_copy(pre.at[idx_g0], gb0, sem0)

                pltpu.make_async_copy(pre.at[idx_g1], gb1, sem1).wait()
                scale_scatter(gb1, vsc1, idx_s1, sem_s1)
                return carry2

            lax.fori_loop(0, INNER // 2, pair_body, 0)
            wait_scatter(gb0, idx_s0, sem_s0)
            wait_scatter(gb1, idx_s1, sem_s1)
            return carry

        lax.fori_loop(0, n_super, super_body, 0)
        writeback(out_h)

    run_phase_a(nrow, ncol, nval, NSUP_A, e_out)


def _sc_body_bc(pre, srow, scol, sval, hrow, hcol, hval,
                tu_out, ti_out,
                rowb, colb, valb, idx_g0, idx_s0, vsc0, idx_g1, idx_s1, vsc1,
                gb0, gb1, zbuf, acc,
                sem0, sem1, sem_s0, sem_s1):
    cid = lax.axis_index("c")
    sid = lax.axis_index("s")
    ii16 = lax.broadcasted_iota(jnp.int32, (16,), 0)
    lanes = [ii16 * 0 + e for e in range(16)]

    def zrow(r, carry):
        for j in range(D // 16):
            zbuf[r, pl.ds(j * 16, 16)] = jnp.zeros((16,), jnp.float32)
        return carry

    lax.fori_loop(0, ZR, zrow, 0)

    def zero_acc():
        zbase = sid * ROWS_T
        for z in range(ROWS_T // ZR):
            pltpu.sync_copy(zbuf, acc.at[pl.ds(zbase + z * ZR, ZR)])
        plsc.subcore_barrier()

    def writeback(out_h):
        plsc.subcore_barrier()
        pltpu.sync_copy(acc.at[pl.ds(sid * ROWS_T, ROWS_T)],
                        out_h.at[cid, pl.ds(sid * ROWS_T, ROWS_T)])
        plsc.subcore_barrier()

    def scale_chunk(gb, vs, voff):
        def scale_body(g, carry3):
            v16 = vs[pl.ds(voff + g * 16, 16)]
            base = g * 16
            for e in range(16):
                bc = v16.at[lanes[e]].get(mode="promise_in_bounds")
                for j in range(D // 16):
                    sl = pl.ds(j * 16, 16)
                    gb[base + e, sl] = gb[base + e, sl] * bc
            return carry3

        lax.fori_loop(0, CH // 16, scale_body, 0)

    def wait_scatter(gb, isc, sem_s):
        pltpu.make_async_copy(gb, acc.at[isc], sem_s).wait()

    def run_phase(row_h, col_h, val_h, n_super, col_off, out_h):
        zero_acc()
        edge_base = (cid * NS + sid) * (n_super * SUP)

        def compute_idx(ci, ig, isc, vs):
            cb = ci * CH
            for j in range(CH // 16):
                sl16 = pl.ds(j * 16, 16)
                slb = pl.ds(cb + j * 16, 16)
                isc[sl16] = rowb[slb]
                ig[sl16] = colb[slb] + col_off
                vs[sl16] = valb[slb]

        def scale_scatter(gb, vs, isc, sem_s):
            scale_chunk(gb, vs, 0)
            pltpu.async_copy(gb, acc.at[isc], sem_s, add=True)

        def super_body(g, carry):
            sb = edge_base + g * SUP
            pltpu.sync_copy(row_h.at[pl.ds(sb, EBUF)], rowb)
            pltpu.sync_copy(col_h.at[pl.ds(sb, EBUF)], colb)
            pltpu.sync_copy(val_h.at[pl.ds(sb, EBUF)], valb)

            compute_idx(0, idx_g0, idx_s0, vsc0)
            pltpu.async_copy(pre.at[idx_g0], gb0, sem0)

            def pair_body(h, carry2):
                @pl.when(h > 0)
                def _():
                    wait_scatter(gb1, idx_s1, sem_s1)

                compute_idx(2 * h + 1, idx_g1, idx_s1, vsc1)
                pltpu.async_copy(pre.at[idx_g1], gb1, sem1)
                pltpu.make_async_copy(pre.at[idx_g0], gb0, sem0).wait()
                scale_scatter(gb0, vsc0, idx_s0, sem_s0)

                @pl.when(h < INNER // 2 - 1)
                def _():
                    wait_scatter(gb0, idx_s0, sem_s0)
                    compute_idx(2 * h + 2, idx_g0, idx_s0, vsc0)
                    pltpu.async_copy(pre.at[idx_g0], gb0, sem0)

                pltpu.make_async_copy(pre.at[idx_g1], gb1, sem1).wait()
                scale_scatter(gb1, vsc1, idx_s1, sem_s1)
                return carry2

            lax.fori_loop(0, INNER // 2, pair_body, 0)
            wait_scatter(gb0, idx_s0, sem_s0)
            wait_scatter(gb1, idx_s1, sem_s1)
            return carry

        lax.fori_loop(0, n_super, super_body, 0)
        writeback(out_h)

    run_phase(srow, scol, sval, NSUP_B, 0, tu_out)
    run_phase(hrow, hcol, hval, NSUP_B, NU, ti_out)


_MESH = plsc.VectorSubcoreMesh(core_axis_name="c", subcore_axis_name="s",
                               num_cores=NC, num_subcores=NS)

_sc_spmm_a = functools.partial(
    pl.kernel,
    out_type=[jax.ShapeDtypeStruct((NC, HP, D), jnp.float32)],
    mesh=_MESH,
    compiler_params=pltpu.CompilerParams(use_tc_tiling_on_sc=False),
    scratch_types=[
        pltpu.VMEM((EBUF,), jnp.int32),
        pltpu.VMEM((EBUF,), jnp.int32),
        pltpu.VMEM((EBUF,), jnp.float32),
        pltpu.VMEM((CH,), jnp.int32),
        pltpu.VMEM((CH,), jnp.int32),
        pltpu.VMEM((CH,), jnp.float32),
        pltpu.VMEM((CH,), jnp.int32),
        pltpu.VMEM((CH,), jnp.int32),
        pltpu.VMEM((CH,), jnp.float32),
        pltpu.VMEM((CH, D), jnp.float32),
        pltpu.VMEM((CH, D), jnp.float32),
        pltpu.VMEM((ZR, D), jnp.float32),
        pltpu.VMEM_SHARED((HP, D), jnp.float32),
        pltpu.SemaphoreType.DMA,
        pltpu.SemaphoreType.DMA,
        pltpu.SemaphoreType.DMA,
        pltpu.SemaphoreType.DMA,
    ],
)(_sc_body_a)

_sc_spmm_bc = functools.partial(
    pl.kernel,
    out_type=[
        jax.ShapeDtypeStruct((NC, HP, D), jnp.float32),
        jax.ShapeDtypeStruct((NC, HP, D), jnp.float32),
    ],
    mesh=_MESH,
    compiler_params=pltpu.CompilerParams(use_tc_tiling_on_sc=False),
    scratch_types=[
        pltpu.VMEM((EBUF,), jnp.int32),
        pltpu.VMEM((EBUF,), jnp.int32),
        pltpu.VMEM((EBUF,), jnp.float32),
        pltpu.VMEM((CH,), jnp.int32),
        pltpu.VMEM((CH,), jnp.int32),
        pltpu.VMEM((CH,), jnp.float32),
        pltpu.VMEM((CH,), jnp.int32),
        pltpu.VMEM((CH,), jnp.int32),
        pltpu.VMEM((CH,), jnp.float32),
        pltpu.VMEM((CH, D), jnp.float32),
        pltpu.VMEM((CH, D), jnp.float32),
        pltpu.VMEM((ZR, D), jnp.float32),
        pltpu.VMEM_SHARED((HP, D), jnp.float32),
        pltpu.SemaphoreType.DMA,
        pltpu.SemaphoreType.DMA,
        pltpu.SemaphoreType.DMA,
        pltpu.SemaphoreType.DMA,
    ],
)(_sc_body_bc)


def _pad_edges(idx, val, total, mod):
    e = val.shape[0]
    p = total - e
    ar = jnp.arange(p, dtype=jnp.int32)
    fill = (ar * 7) % mod
    row = jnp.concatenate([idx[0], fill])
    col = jnp.concatenate([idx[1], fill])
    valp = jnp.concatenate([val, jnp.zeros((p,), val.dtype)])
    return row, col, valp


def _g_body(e_ref, emb_ref, qu_ref, qi_ref, w1u_ref, w2u_ref, bu_ref,
            w1i_ref, w2i_ref, bi_ref, g_ref):
    is_u = pl.program_id(0) < (NU // _BT)
    q = jnp.where(is_u, qu_ref[...], qi_ref[...])
    w1 = jnp.where(is_u, w1u_ref[...], w1i_ref[...])
    w2 = jnp.where(is_u, w2u_ref[...], w2i_ref[...])
    b = jnp.where(is_u, bu_ref[...], bi_ref[...])
    t = jnp.tanh(e_ref[0] @ q)
    g = jnp.tanh(emb_ref[...] @ w1 + t @ w2 + b)
    n = jnp.sqrt(jnp.sum(g * g, axis=1, keepdims=True))
    g_ref[...] = g / jnp.maximum(n, 1e-12)


def _out_body(g_ref, tu0_ref, tu1_ref, ti0_ref, ti1_ref, mu_ref, mi_ref,
              wm_ref, bm_ref, o_ref):
    is_u = pl.program_id(0) < (NU // _BT)
    m = jnp.where(is_u, mu_ref[...], mi_ref[...])
    t = jnp.where(is_u, tu0_ref[0] + tu1_ref[0], ti0_ref[0] + ti1_ref[0])
    ug = g_ref[...] + jnp.tanh(t @ m)
    mlp = jnp.tanh(ug @ wm_ref[...] + bm_ref[...])
    o_ref[...] = jnp.where(is_u, mlp, ug)


_BT = 1000


def _half_spec():
    nb = NU // _BT
    return pl.BlockSpec((1, _BT, D), lambda i: (i // nb, i % nb, 0))


def _core_spec(c):
    nb = NU // _BT
    return pl.BlockSpec((1, _BT, D), lambda i, c=c: (c, i % nb, 0))


def _blk_spec():
    return pl.BlockSpec((_BT, D), lambda i: (i, 0))


def _full(shape):
    return pl.BlockSpec(shape, lambda i: tuple(0 for _ in shape))


def kernel(user_emb, item_emb, norm_idx, norm_val, sym_idx, sym_val,
           herb_idx, herb_val, Q_user, W_gc_user, b_gc_user, Q_item,
           W_gc_item, b_gc_item, M_user, M_item, W_mlp_user, b_mlp_user):
    pre = jnp.concatenate([user_emb, item_emb], axis=0)
    nrow, ncol, nval = _pad_edges(norm_idx, norm_val, EA + 272, NN)
    srow, scol, sval = _pad_edges(sym_idx, sym_val, EB + 272, NU)
    hrow, hcol, hval = _pad_edges(herb_idx, herb_val, EB + 272, NI)

    (e2,) = _sc_spmm_a(pre, nrow, ncol, nval)
    tup, tip = _sc_spmm_bc(pre, srow, scol, sval, hrow, hcol, hval)

    w1u, w2u = W_gc_user[:D], W_gc_user[D:]
    w1i, w2i = W_gc_item[:D], W_gc_item[D:]
    grid = (NN // _BT,)

    g = pl.pallas_call(
        _g_body,
        grid=grid,
        in_specs=[
            _half_spec(),
            _blk_spec(),
            _full((D, D)), _full((D, D)),
            _full((D, D)), _full((D, D)), _full((1, D)),
            _full((D, D)), _full((D, D)), _full((1, D)),
        ],
        out_specs=_blk_spec(),
        out_shape=jax.ShapeDtypeStruct((NN, D), jnp.float32),
    )(e2, pre, Q_user, Q_item, w1u, w2u, b_gc_user, w1i, w2i, b_gc_item)

    out = pl.pallas_call(
        _out_body,
        grid=grid,
        in_specs=[
            _blk_spec(),
            _core_spec(0), _core_spec(1),
            _core_spec(0), _core_spec(1),
            _full((D, D)), _full((D, D)),
            _full((D, D)), _full((1, D)),
        ],
        out_specs=_blk_spec(),
        out_shape=jax.ShapeDtypeStruct((NN, D), jnp.float32),
    )(g, tup, tup, tip, tip, M_user, M_item, W_mlp_user, b_mlp_user)

    return out


# speedup vs baseline: 1.2850x; 1.2850x over previous
import functools

import jax
import jax.numpy as jnp
from jax import lax
from jax.experimental import pallas as pl
from jax.experimental.pallas import tpu as pltpu
from jax.experimental.pallas import tpu_sc as plsc

NU = 25000
NI = 25000
NN = NU + NI
D = 64
NC = 2
NS = 16
H = 25000
HP = 25088
ROWS_T = HP // NS
CH = 128
INNER = 14
SUP = CH * INNER
NSUP_A = 28
NSUP_B = 7
EA = NS * NSUP_A * SUP
EB = NC * NS * NSUP_B * SUP
ZR = 28
EBUF = SUP + 272


def _sc_body(pre, nrow, ncol, nval, srow, scol, sval, hrow, hcol, hval,
             e_out, tu_out, ti_out,
             rowb, colb, valb, idx_g0, idx_s0, vsc0, idx_g1, idx_s1, vsc1,
             gb0, gb1, zbuf, acc,
             sem0, sem1, sem_s0, sem_s1):
    cid = lax.axis_index("c")
    sid = lax.axis_index("s")
    ii16 = lax.broadcasted_iota(jnp.int32, (16,), 0)
    lanes = [ii16 * 0 + e for e in range(16)]

    def zrow(r, carry):
        for j in range(D // 16):
            zbuf[r, pl.ds(j * 16, 16)] = jnp.zeros((16,), jnp.float32)
        return carry

    lax.fori_loop(0, ZR, zrow, 0)

    def zero_acc():
        zbase = sid * ROWS_T
        for z in range(ROWS_T // ZR):
            pltpu.sync_copy(zbuf, acc.at[pl.ds(zbase + z * ZR, ZR)])
        plsc.subcore_barrier()

    def writeback(out_h):
        plsc.subcore_barrier()
        pltpu.sync_copy(acc.at[pl.ds(sid * ROWS_T, ROWS_T)],
                        out_h.at[cid, pl.ds(sid * ROWS_T, ROWS_T)])
        plsc.subcore_barrier()

    def scale_chunk(gb, vs, voff):
        def scale_body(g, carry3):
            v16 = vs[pl.ds(voff + g * 16, 16)]
            base = g * 16
            for e in range(16):
                bc = v16.at[lanes[e]].get(mode="promise_in_bounds")
                for j in range(D // 16):
                    sl = pl.ds(j * 16, 16)
                    gb[base + e, sl] = gb[base + e, sl] * bc
            return carry3

        lax.fori_loop(0, CH // 16, scale_body, 0)

    def wait_scatter(gb, isc, sem_s):
        pltpu.make_async_copy(gb, acc.at[isc], sem_s).wait()

    def run_phase_a(row_h, col_h, val_h, n_super, out_h):
        zero_acc()
        edge_base = sid * (n_super * SUP)
        lo = cid * H
        shidx = [jnp.maximum(ii16 - k, 0) for k in (1, 2, 4, 8)]
        shmask = [ii16 >= k for k in (1, 2, 4, 8)]

        def prep_idx(isc, ig, off):
            for j in range(CH // 16):
                isc[pl.ds(j * 16, 16)] = rowb[pl.ds(off + j * 16, 16)]
                ig[pl.ds(j * 16, 16)] = colb[pl.ds(off + j * 16, 16)]

        def super_body(g, carry):
            sb = edge_base + g * SUP
            pltpu.sync_copy(row_h.at[pl.ds(sb, EBUF)], rowb)
            pltpu.sync_copy(col_h.at[pl.ds(sb, EBUF)], colb)
            pltpu.sync_copy(val_h.at[pl.ds(sb, EBUF)], valb)

            def fbody(j, p):
                slb = pl.ds(j * 16, 16)
                r = rowb[slb]
                cc = colb[slb]
                vv = valb[slb]
                m = r - jnp.where(r >= H, H, 0)
                ok = (r >= lo) & (r < lo + H)
                s = jnp.where(ok, 1, 0)
                for t in range(4):
                    sh = s.at[shidx[t]].get(mode="promise_in_bounds")
                    s = s + jnp.where(shmask[t], sh, 0)
                sel = ii16 * 0
                for step in (8, 4, 2, 1):
                    sv = s.at[sel + (step - 1)].get(mode="promise_in_bounds")
                    sel = jnp.where(sv < ii16 + 1, sel + step, sel)
                slw = pl.ds(p, 16)
                rowb[slw] = m.at[sel].get(mode="promise_in_bounds")
                colb[slw] = cc.at[sel].get(mode="promise_in_bounds")
                valb[slw] = vv.at[sel].get(mode="promise_in_bounds")
                return p + s[15]

            p = lax.fori_loop(0, SUP // 16, fbody, 0)
            for j in range(16):
                slp = pl.ds(p + j * 16, 16)
                rowb[slp] = ii16 + (j * 16)
                colb[slp] = ii16 + (j * 16)
                valb[slp] = jnp.zeros((16,), jnp.float32)
            npair = (p + 255) // 256

            @pl.when(npair > 0)
            def _():
                prep_idx(idx_s0, idx_g0, 0)
                pltpu.async_copy(pre.at[idx_g0], gb0, sem0)

            def pair_body(i, carry2):
                c0 = i * 256
                c1 = c0 + CH

                @pl.when(i > 0)
                def _():
                    wait_scatter(gb1, idx_s1, sem_s1)

                prep_idx(idx_s1, idx_g1, c1)
                pltpu.async_copy(pre.at[idx_g1], gb1, sem1)
                pltpu.make_async_copy(pre.at[idx_g0], gb0, sem0).wait()
                scale_chunk(gb0, valb, c0)
                pltpu.async_copy(gb0, acc.at[idx_s0], sem_s0, add=True)

                @pl.when(i < npair - 1)
                def _():
                    wait_scatter(gb0, idx_s0, sem_s0)
                    prep_idx(idx_s0, idx_g0, c0 + 256)
                    pltpu.async_copy(pre.at[idx_g0], gb0, sem0)

                pltpu.make_async_copy(pre.at[idx_g1], gb1, sem1).wait()
                scale_chunk(gb1, valb, c1)
                pltpu.async_copy(gb1, acc.at[idx_s1], sem_s1, add=True)
                return carry2

            lax.fori_loop(0, npair, pair_body, 0)

            @pl.when(npair > 0)
            def _():
                wait_scatter(gb0, idx_s0, sem_s0)
                wait_scatter(gb1, idx_s1, sem_s1)

            return carry

        lax.fori_loop(0, n_super, super_body, 0)
        writeback(out_h)

    def run_phase(row_h, col_h, val_h, n_super, col_off, out_h):
        zero_acc()
        edge_base = (cid * NS + sid) * (n_super * SUP)

        def compute_idx(ci, ig, isc, vs):
            cb = ci * CH
            for j in range(CH // 16):
                sl16 = pl.ds(j * 16, 16)
                slb = pl.ds(cb + j * 16, 16)
                isc[sl16] = rowb[slb]
                ig[sl16] = colb[slb] + col_off
                vs[sl16] = valb[slb]

        def scale_scatter(gb, vs, isc, sem_s):
            scale_chunk(gb, vs, 0)
            pltpu.async_copy(gb, acc.at[isc], sem_s, add=True)

        def super_body(g, carry):
            sb = edge_base + g * SUP
            pltpu.sync_copy(row_h.at[pl.ds(sb, EBUF)], rowb)
            pltpu.sync_copy(col_h.at[pl.ds(sb, EBUF)], colb)
            pltpu.sync_copy(val_h.at[pl.ds(sb, EBUF)], valb)

            compute_idx(0, idx_g0, idx_s0, vsc0)
            pltpu.async_copy(pre.at[idx_g0], gb0, sem0)

            def pair_body(h, carry2):
                @pl.when(h > 0)
                def _():
                    wait_scatter(gb1, idx_s1, sem_s1)

                compute_idx(2 * h + 1, idx_g1, idx_s1, vsc1)
                pltpu.async_copy(pre.at[idx_g1], gb1, sem1)
                pltpu.make_async_copy(pre.at[idx_g0], gb0, sem0).wait()
                scale_scatter(gb0, vsc0, idx_s0, sem_s0)

                @pl.when(h < INNER // 2 - 1)
                def _():
                    wait_scatter(gb0, idx_s0, sem_s0)
                    compute_idx(2 * h + 2, idx_g0, idx_s0, vsc0)
                    pltpu.async_copy(pre.at[idx_g0], gb0, sem0)

                pltpu.make_async_copy(pre.at[idx_g1], gb1, sem1).wait()
                scale_scatter(gb1, vsc1, idx_s1, sem_s1)
                return carry2

            lax.fori_loop(0, INNER // 2, pair_body, 0)
            wait_scatter(gb0, idx_s0, sem_s0)
            wait_scatter(gb1, idx_s1, sem_s1)
            return carry

        lax.fori_loop(0, n_super, super_body, 0)
        writeback(out_h)

    run_phase_a(nrow, ncol, nval, NSUP_A, e_out)
    run_phase(srow, scol, sval, NSUP_B, 0, tu_out)
    run_phase(hrow, hcol, hval, NSUP_B, NU, ti_out)


_sc_spmm = functools.partial(
    pl.kernel,
    out_type=[
        jax.ShapeDtypeStruct((NC, HP, D), jnp.float32),
        jax.ShapeDtypeStruct((NC, HP, D), jnp.float32),
        jax.ShapeDtypeStruct((NC, HP, D), jnp.float32),
    ],
    mesh=plsc.VectorSubcoreMesh(
        core_axis_name="c", subcore_axis_name="s",
        num_cores=NC, num_subcores=NS),
    compiler_params=pltpu.CompilerParams(use_tc_tiling_on_sc=False),
    scratch_types=[
        pltpu.VMEM((EBUF,), jnp.int32),
        pltpu.VMEM((EBUF,), jnp.int32),
        pltpu.VMEM((EBUF,), jnp.float32),
        pltpu.VMEM((CH,), jnp.int32),
        pltpu.VMEM((CH,), jnp.int32),
        pltpu.VMEM((CH,), jnp.float32),
        pltpu.VMEM((CH,), jnp.int32),
        pltpu.VMEM((CH,), jnp.int32),
        pltpu.VMEM((CH,), jnp.float32),
        pltpu.VMEM((CH, D), jnp.float32),
        pltpu.VMEM((CH, D), jnp.float32),
        pltpu.VMEM((ZR, D), jnp.float32),
        pltpu.VMEM_SHARED((HP, D), jnp.float32),
        pltpu.SemaphoreType.DMA,
        pltpu.SemaphoreType.DMA,
        pltpu.SemaphoreType.DMA,
        pltpu.SemaphoreType.DMA,
    ],
)(_sc_body)


def _pad_edges(idx, val, total, mod):
    e = val.shape[0]
    p = total - e
    ar = jnp.arange(p, dtype=jnp.int32)
    fill = (ar * 7) % mod
    row = jnp.concatenate([idx[0], fill])
    col = jnp.concatenate([idx[1], fill])
    valp = jnp.concatenate([val, jnp.zeros((p,), val.dtype)])
    return row, col, valp


def _dense_body(e_ref, emb_ref, tu0_ref, tu1_ref, ti0_ref, ti1_ref,
                qu_ref, qi_ref, w1u_ref, w2u_ref, bu_ref,
                w1i_ref, w2i_ref, bi_ref,
                mu_ref, mi_ref, wm_ref, bm_ref, o_ref):
    is_u = pl.program_id(0) < (NU // _BT)
    q = jnp.where(is_u, qu_ref[...], qi_ref[...])
    w1 = jnp.where(is_u, w1u_ref[...], w1i_ref[...])
    w2 = jnp.where(is_u, w2u_ref[...], w2i_ref[...])
    b = jnp.where(is_u, bu_ref[...], bi_ref[...])
    m = jnp.where(is_u, mu_ref[...], mi_ref[...])
    t = jnp.tanh(e_ref[0] @ q)
    g = jnp.tanh(emb_ref[...] @ w1 + t @ w2 + b)
    n = jnp.sqrt(jnp.sum(g * g, axis=1, keepdims=True))
    g = g / jnp.maximum(n, 1e-12)
    t = jnp.where(is_u, tu0_ref[0] + tu1_ref[0], ti0_ref[0] + ti1_ref[0])
    ug = g + jnp.tanh(t @ m)
    mlp = jnp.tanh(ug @ wm_ref[...] + bm_ref[...])
    o_ref[...] = jnp.where(is_u, mlp, ug)


_BT = 1000


def _half_spec():
    nb = NU // _BT
    return pl.BlockSpec((1, _BT, D), lambda i: (i // nb, i % nb, 0))


def _hpair_spec(tu, ti, c):
    del tu, ti
    nb = NU // _BT
    return pl.BlockSpec((1, _BT, D), lambda i, c=c: (c, i % nb, 0))


def _blk_spec():
    return pl.BlockSpec((_BT, D), lambda i: (i, 0))


def _full(shape):
    return pl.BlockSpec(shape, lambda i: tuple(0 for _ in shape))


def kernel(user_emb, item_emb, norm_idx, norm_val, sym_idx, sym_val,
           herb_idx, herb_val, Q_user, W_gc_user, b_gc_user, Q_item,
           W_gc_item, b_gc_item, M_user, M_item, W_mlp_user, b_mlp_user):
    pre = jnp.concatenate([user_emb, item_emb], axis=0)
    nrow, ncol, nval = _pad_edges(norm_idx, norm_val, EA + 272, NN)
    srow, scol, sval = _pad_edges(sym_idx, sym_val, EB + 272, NU)
    hrow, hcol, hval = _pad_edges(herb_idx, herb_val, EB + 272, NI)

    e2, tup, tip = _sc_spmm(pre, nrow, ncol, nval, srow, scol, sval,
                            hrow, hcol, hval)

    w1u, w2u = W_gc_user[:D], W_gc_user[D:]
    w1i, w2i = W_gc_item[:D], W_gc_item[D:]
    nb = NU // _BT

    def tsel(c):
        return pl.BlockSpec((1, _BT, D), lambda i, c=c: (c, i % nb, 0))

    out = pl.pallas_call(
        _dense_body,
        grid=(NN // _BT,),
        in_specs=[
            _half_spec(),
            _blk_spec(),
            tsel(0), tsel(1),
            tsel(0), tsel(1),
            _full((D, D)), _full((D, D)),
            _full((D, D)), _full((D, D)), _full((1, D)),
            _full((D, D)), _full((D, D)), _full((1, D)),
            _full((D, D)), _full((D, D)),
            _full((D, D)), _full((1, D)),
        ],
        out_specs=_blk_spec(),
        out_shape=jax.ShapeDtypeStruct((NN, D), jnp.float32),
    )(e2, pre, tup, tup, tip, tip, Q_user, Q_item, w1u, w2u, b_gc_user,
      w1i, w2i, b_gc_item, M_user, M_item, W_mlp_user, b_mlp_user)

    return out


# speedup vs baseline: 1.3416x; 1.0441x over previous
import functools

import jax
import jax.numpy as jnp
from jax import lax
from jax.experimental import pallas as pl
from jax.experimental.pallas import tpu as pltpu
from jax.experimental.pallas import tpu_sc as plsc

NU = 25000
NI = 25000
NN = NU + NI
D = 64
NC = 2
NS = 16
H = 25000
HP = 25088
ROWS_T = HP // NS
CH = 128
INNER = 14
SUP = CH * INNER
NSUP_A = 28
NSUP_B = 7
EA = NS * NSUP_A * SUP
EB = NC * NS * NSUP_B * SUP
ZR = 16
EBUF = SUP + 272


def _sc_body(pre, nrow, ncol, nval, srow, scol, sval, hrow, hcol, hval,
             e_out, tu_out, ti_out,
             rowb, colb, valb, rowc, colc, valc,
             idx_g0, idx_s0, vsc0, idx_g1, idx_s1, vsc1,
             gb0, gb1, zbuf, acc,
             sem0, sem1, sem_s0, sem_s1, sem_e0, sem_e1):
    cid = lax.axis_index("c")
    sid = lax.axis_index("s")
    ii16 = lax.broadcasted_iota(jnp.int32, (16,), 0)
    lanes = [ii16 * 0 + e for e in range(16)]

    def zrow(r, carry):
        for j in range(D // 16):
            zbuf[r, pl.ds(j * 16, 16)] = jnp.zeros((16,), jnp.float32)
        return carry

    lax.fori_loop(0, ZR, zrow, 0)

    def zero_acc():
        zbase = sid * ROWS_T
        for z in range(ROWS_T // ZR):
            pltpu.sync_copy(zbuf, acc.at[pl.ds(zbase + z * ZR, ZR)])
        plsc.subcore_barrier()

    def writeback(out_h):
        plsc.subcore_barrier()
        pltpu.sync_copy(acc.at[pl.ds(sid * ROWS_T, ROWS_T)],
                        out_h.at[cid, pl.ds(sid * ROWS_T, ROWS_T)])
        plsc.subcore_barrier()

    def scale_chunk(gb, vs, voff):
        def scale_body(g, carry3):
            v16 = vs[pl.ds(voff + g * 16, 16)]
            base = g * 16
            for e in range(16):
                bc = v16.at[lanes[e]].get(mode="promise_in_bounds")
                for j in range(D // 16):
                    sl = pl.ds(j * 16, 16)
                    gb[base + e, sl] = gb[base + e, sl] * bc
            return carry3

        lax.fori_loop(0, CH // 16, scale_body, 0)

    def wait_scatter(gb, isc, sem_s):
        pltpu.make_async_copy(gb, acc.at[isc], sem_s).wait()

    def run_phase_a(row_h, col_h, val_h, n_super, out_h):
        zero_acc()
        edge_base = sid * (n_super * SUP)
        lo = cid * H
        shidx = [jnp.maximum(ii16 - k, 0) for k in (1, 2, 4, 8)]
        shmask = [ii16 >= k for k in (1, 2, 4, 8)]

        def load_edges(rb, cb, vb, g, sem):
            sb = edge_base + g * SUP
            pltpu.async_copy(row_h.at[pl.ds(sb, EBUF)], rb, sem)
            pltpu.async_copy(col_h.at[pl.ds(sb, EBUF)], cb, sem)
            pltpu.async_copy(val_h.at[pl.ds(sb, EBUF)], vb, sem)

        def wait_edges(rb, cb, vb, g, sem):
            sb = edge_base + g * SUP
            pltpu.make_async_copy(row_h.at[pl.ds(sb, EBUF)], rb, sem).wait()
            pltpu.make_async_copy(col_h.at[pl.ds(sb, EBUF)], cb, sem).wait()
            pltpu.make_async_copy(val_h.at[pl.ds(sb, EBUF)], vb, sem).wait()

        def process_super(rb, cb, vb):
            def prep_idx(isc, ig, off):
                for j in range(CH // 16):
                    isc[pl.ds(j * 16, 16)] = rb[pl.ds(off + j * 16, 16)]
                    ig[pl.ds(j * 16, 16)] = cb[pl.ds(off + j * 16, 16)]

            def fbody(j, p):
                slb = pl.ds(j * 16, 16)
                r = rb[slb]
                cc = cb[slb]
                vv = vb[slb]
                m = r - jnp.where(r >= H, H, 0)
                ok = (r >= lo) & (r < lo + H)
                s = jnp.where(ok, 1, 0)
                for t in range(4):
                    sh = s.at[shidx[t]].get(mode="promise_in_bounds")
                    s = s + jnp.where(shmask[t], sh, 0)
                sel = ii16 * 0
                for step in (8, 4, 2, 1):
                    sv = s.at[sel + (step - 1)].get(mode="promise_in_bounds")
                    sel = jnp.where(sv < ii16 + 1, sel + step, sel)
                slw = pl.ds(p, 16)
                rb[slw] = m.at[sel].get(mode="promise_in_bounds")
                cb[slw] = cc.at[sel].get(mode="promise_in_bounds")
                vb[slw] = vv.at[sel].get(mode="promise_in_bounds")
                return p + s[15]

            p = lax.fori_loop(0, SUP // 16, fbody, 0)
            for j in range(16):
                slp = pl.ds(p + j * 16, 16)
                rb[slp] = ii16 + (j * 16)
                cb[slp] = ii16 + (j * 16)
                vb[slp] = jnp.zeros((16,), jnp.float32)
            npair = (p + 255) // 256

            @pl.when(npair > 0)
            def _():
                prep_idx(idx_s0, idx_g0, 0)
                pltpu.async_copy(pre.at[idx_g0], gb0, sem0)

            def pair_body(i, carry2):
                c0 = i * 256
                c1 = c0 + CH

                @pl.when(i > 0)
                def _():
                    wait_scatter(gb1, idx_s1, sem_s1)

                prep_idx(idx_s1, idx_g1, c1)
                pltpu.async_copy(pre.at[idx_g1], gb1, sem1)
                pltpu.make_async_copy(pre.at[idx_g0], gb0, sem0).wait()
                scale_chunk(gb0, vb, c0)
                pltpu.async_copy(gb0, acc.at[idx_s0], sem_s0, add=True)

                @pl.when(i < npair - 1)
                def _():
                    wait_scatter(gb0, idx_s0, sem_s0)
                    prep_idx(idx_s0, idx_g0, c0 + 256)
                    pltpu.async_copy(pre.at[idx_g0], gb0, sem0)

                pltpu.make_async_copy(pre.at[idx_g1], gb1, sem1).wait()
                scale_chunk(gb1, vb, c1)
                pltpu.async_copy(gb1, acc.at[idx_s1], sem_s1, add=True)
                return carry2

            lax.fori_loop(0, npair, pair_body, 0)

            @pl.when(npair > 0)
            def _():
                wait_scatter(gb0, idx_s0, sem_s0)
                wait_scatter(gb1, idx_s1, sem_s1)

        load_edges(rowb, colb, valb, 0, sem_e0)

        def pair_super(hg, carry):
            g0 = 2 * hg
            load_edges(rowc, colc, valc, g0 + 1, sem_e1)
            wait_edges(rowb, colb, valb, g0, sem_e0)
            process_super(rowb, colb, valb)

            @pl.when(hg < n_super // 2 - 1)
            def _():
                load_edges(rowb, colb, valb, g0 + 2, sem_e0)

            wait_edges(rowc, colc, valc, g0 + 1, sem_e1)
            process_super(rowc, colc, valc)
            return carry

        lax.fori_loop(0, n_super // 2, pair_super, 0)
        writeback(out_h)

    def run_phase(row_h, col_h, val_h, n_super, col_off, out_h):
        zero_acc()
        edge_base = (cid * NS + sid) * (n_super * SUP)

        def compute_idx(ci, ig, isc, vs):
            cb = ci * CH
            for j in range(CH // 16):
                sl16 = pl.ds(j * 16, 16)
                slb = pl.ds(cb + j * 16, 16)
                isc[sl16] = rowb[slb]
                ig[sl16] = colb[slb] + col_off
                vs[sl16] = valb[slb]

        def scale_scatter(gb, vs, isc, sem_s):
            scale_chunk(gb, vs, 0)
            pltpu.async_copy(gb, acc.at[isc], sem_s, add=True)

        def super_body(g, carry):
            sb = edge_base + g * SUP
            pltpu.sync_copy(row_h.at[pl.ds(sb, EBUF)], rowb)
            pltpu.sync_copy(col_h.at[pl.ds(sb, EBUF)], colb)
            pltpu.sync_copy(val_h.at[pl.ds(sb, EBUF)], valb)

            compute_idx(0, idx_g0, idx_s0, vsc0)
            pltpu.async_copy(pre.at[idx_g0], gb0, sem0)

            def pair_body(h, carry2):
                @pl.when(h > 0)
                def _():
                    wait_scatter(gb1, idx_s1, sem_s1)

                compute_idx(2 * h + 1, idx_g1, idx_s1, vsc1)
                pltpu.async_copy(pre.at[idx_g1], gb1, sem1)
                pltpu.make_async_copy(pre.at[idx_g0], gb0, sem0).wait()
                scale_scatter(gb0, vsc0, idx_s0, sem_s0)

                @pl.when(h < INNER // 2 - 1)
                def _():
                    wait_scatter(gb0, idx_s0, sem_s0)
                    compute_idx(2 * h + 2, idx_g0, idx_s0, vsc0)
                    pltpu.async_copy(pre.at[idx_g0], gb0, sem0)

                pltpu.make_async_copy(pre.at[idx_g1], gb1, sem1).wait()
                scale_scatter(gb1, vsc1, idx_s1, sem_s1)
                return carry2

            lax.fori_loop(0, INNER // 2, pair_body, 0)
            wait_scatter(gb0, idx_s0, sem_s0)
            wait_scatter(gb1, idx_s1, sem_s1)
            return carry

        lax.fori_loop(0, n_super, super_body, 0)
        writeback(out_h)

    run_phase_a(nrow, ncol, nval, NSUP_A, e_out)
    run_phase(srow, scol, sval, NSUP_B, 0, tu_out)
    run_phase(hrow, hcol, hval, NSUP_B, NU, ti_out)


_sc_spmm = functools.partial(
    pl.kernel,
    out_type=[
        jax.ShapeDtypeStruct((NC, HP, D), jnp.float32),
        jax.ShapeDtypeStruct((NC, HP, D), jnp.float32),
        jax.ShapeDtypeStruct((NC, HP, D), jnp.float32),
    ],
    mesh=plsc.VectorSubcoreMesh(
        core_axis_name="c", subcore_axis_name="s",
        num_cores=NC, num_subcores=NS),
    compiler_params=pltpu.CompilerParams(use_tc_tiling_on_sc=False),
    scratch_types=[
        pltpu.VMEM((EBUF,), jnp.int32),
        pltpu.VMEM((EBUF,), jnp.int32),
        pltpu.VMEM((EBUF,), jnp.float32),
        pltpu.VMEM((EBUF,), jnp.int32),
        pltpu.VMEM((EBUF,), jnp.int32),
        pltpu.VMEM((EBUF,), jnp.float32),
        pltpu.VMEM((CH,), jnp.int32),
        pltpu.VMEM((CH,), jnp.int32),
        pltpu.VMEM((CH,), jnp.float32),
        pltpu.VMEM((CH,), jnp.int32),
        pltpu.VMEM((CH,), jnp.int32),
        pltpu.VMEM((CH,), jnp.float32),
        pltpu.VMEM((CH, D), jnp.float32),
        pltpu.VMEM((CH, D), jnp.float32),
        pltpu.VMEM((ZR, D), jnp.float32),
        pltpu.VMEM_SHARED((HP, D), jnp.float32),
        pltpu.SemaphoreType.DMA,
        pltpu.SemaphoreType.DMA,
        pltpu.SemaphoreType.DMA,
        pltpu.SemaphoreType.DMA,
        pltpu.SemaphoreType.DMA,
        pltpu.SemaphoreType.DMA,
    ],
)(_sc_body)


def _pad_edges(idx, val, total, mod):
    e = val.shape[0]
    p = total - e
    ar = jnp.arange(p, dtype=jnp.int32)
    fill = (ar * 7) % mod
    row = jnp.concatenate([idx[0], fill])
    col = jnp.concatenate([idx[1], fill])
    valp = jnp.concatenate([val, jnp.zeros((p,), val.dtype)])
    return row, col, valp


def _dense_body(e_ref, emb_ref, tu0_ref, tu1_ref, ti0_ref, ti1_ref,
                qu_ref, qi_ref, w1u_ref, w2u_ref, bu_ref,
                w1i_ref, w2i_ref, bi_ref,
                mu_ref, mi_ref, wm_ref, bm_ref, o_ref):
    is_u = pl.program_id(0) < (NU // _BT)
    q = jnp.where(is_u, qu_ref[...], qi_ref[...])
    w1 = jnp.where(is_u, w1u_ref[...], w1i_ref[...])
    w2 = jnp.where(is_u, w2u_ref[...], w2i_ref[...])
    b = jnp.where(is_u, bu_ref[...], bi_ref[...])
    m = jnp.where(is_u, mu_ref[...], mi_ref[...])
    t = jnp.tanh(e_ref[0] @ q)
    g = jnp.tanh(emb_ref[...] @ w1 + t @ w2 + b)
    n = jnp.sqrt(jnp.sum(g * g, axis=1, keepdims=True))
    g = g / jnp.maximum(n, 1e-12)
    t = jnp.where(is_u, tu0_ref[0] + tu1_ref[0], ti0_ref[0] + ti1_ref[0])
    ug = g + jnp.tanh(t @ m)
    mlp = jnp.tanh(ug @ wm_ref[...] + bm_ref[...])
    o_ref[...] = jnp.where(is_u, mlp, ug)


_BT = 1000


def _half_spec():
    nb = NU // _BT
    return pl.BlockSpec((1, _BT, D), lambda i: (i // nb, i % nb, 0))


def _hpair_spec(tu, ti, c):
    del tu, ti
    nb = NU // _BT
    return pl.BlockSpec((1, _BT, D), lambda i, c=c: (c, i % nb, 0))


def _blk_spec():
    return pl.BlockSpec((_BT, D), lambda i: (i, 0))


def _full(shape):
    return pl.BlockSpec(shape, lambda i: tuple(0 for _ in shape))


def kernel(user_emb, item_emb, norm_idx, norm_val, sym_idx, sym_val,
           herb_idx, herb_val, Q_user, W_gc_user, b_gc_user, Q_item,
           W_gc_item, b_gc_item, M_user, M_item, W_mlp_user, b_mlp_user):
    pre = jnp.concatenate([user_emb, item_emb], axis=0)
    nrow, ncol, nval = _pad_edges(norm_idx, norm_val, EA + 272, NN)
    srow, scol, sval = _pad_edges(sym_idx, sym_val, EB + 272, NU)
    hrow, hcol, hval = _pad_edges(herb_idx, herb_val, EB + 272, NI)

    e2, tup, tip = _sc_spmm(pre, nrow, ncol, nval, srow, scol, sval,
                            hrow, hcol, hval)

    w1u, w2u = W_gc_user[:D], W_gc_user[D:]
    w1i, w2i = W_gc_item[:D], W_gc_item[D:]
    nb = NU // _BT

    def tsel(c):
        return pl.BlockSpec((1, _BT, D), lambda i, c=c: (c, i % nb, 0))

    out = pl.pallas_call(
        _dense_body,
        grid=(NN // _BT,),
        in_specs=[
            _half_spec(),
            _blk_spec(),
            tsel(0), tsel(1),
            tsel(0), tsel(1),
            _full((D, D)), _full((D, D)),
            _full((D, D)), _full((D, D)), _full((1, D)),
            _full((D, D)), _full((D, D)), _full((1, D)),
            _full((D, D)), _full((D, D)),
            _full((D, D)), _full((1, D)),
        ],
        out_specs=_blk_spec(),
        out_shape=jax.ShapeDtypeStruct((NN, D), jnp.float32),
    )(e2, pre, tup, tup, tip, tip, Q_user, Q_item, w1u, w2u, b_gc_user,
      w1i, w2i, b_gc_item, M_user, M_item, W_mlp_user, b_mlp_user)

    return out
